# Initial kernel scaffold; baseline (speedup 1.0000x reference)
import jax
import jax.numpy as jnp
from jax.experimental import pallas as pl


def kernel(z, a):
    raise NotImplementedError("write your pallas kernel here")


# speedup vs baseline: 2.0929x; 2.0929x over previous
import functools

import jax
import jax.numpy as jnp
from jax import lax
from jax.experimental import pallas as pl
from jax.experimental.pallas import tpu as pltpu
from jax.experimental.pallas import tpu_sc as plsc

B = 16384
K = 128
ZC = 26
L = 16


@functools.lru_cache(maxsize=None)
def _build(nc: int, ns: int):
    nw = nc * ns
    bpw = B // nw
    n_vec = bpw // L
    vpr = K // L
    n_dma = bpw // K

    @functools.partial(
        pl.kernel,
        mesh=plsc.VectorSubcoreMesh(core_axis_name="c", subcore_axis_name="s"),
        out_type=jax.ShapeDtypeStruct((nw, n_dma, K), jnp.float32),
        compiler_params=pltpu.CompilerParams(needs_layout_passes=False),
        scratch_types=[
            pltpu.VMEM((bpw * ZC,), jnp.float32),
            pltpu.VMEM((n_dma, K), jnp.int32),
            pltpu.VMEM((n_dma, K), jnp.float32),
            pltpu.SemaphoreType.DMA,
        ],
    )
    def sc_gather(z_hbm, a_hbm, out_hbm, zv, idxv, valsv, sem):
        wid = lax.axis_index("s") * nc + lax.axis_index("c")
        base = wid * bpw

        pltpu.sync_copy(z_hbm.at[pl.ds(base * ZC, bpw * ZC)], zv)

        for i in range(n_vec):
            r16 = lax.iota(jnp.int32, L) + (i * L)
            slot = plsc.load_gather(zv, [r16 * ZC])
            fl = (base + r16) * K + slot.astype(jnp.int32)
            idxv[i // vpr, pl.ds((i % vpr) * L, L)] = fl

        copies = [
            pltpu.async_copy(a_hbm.at[idxv.at[j]], valsv.at[j], sem)
            for j in range(n_dma)
        ]
        for c in copies:
            c.wait()

        for i in range(n_vec):
            j, s = divmod(i, vpr)
            valsv[j, pl.ds(s * L, L)] = (
                valsv[j, pl.ds(s * L, L)] * jnp.float32(0.999)
            )

        pltpu.sync_copy(valsv, out_hbm.at[wid])

    return sc_gather


def kernel(z, a):
    info = plsc.get_sparse_core_info()
    out = _build(info.num_cores, info.num_subcores)(z.reshape(-1), a.reshape(-1))
    return out.reshape(B)
